# baseline (speedup 1.0000x reference)
import functools
import math

import jax
import jax.numpy as jnp
from jax import lax
from jax.experimental import pallas as pl
from jax.experimental.pallas import tpu as pltpu
from jax.experimental.pallas import tpu_sc as plsc

_D = 128
_NC = 2
_NS = 16
_NW = _NC * _NS
_C = 80
_ZR = 8


def _round_up(x, m):
    return (x + m - 1) // m * m


@functools.cache
def _make_seg_sum(n_pad, num_edges):
    assert num_edges % (_NW * _C) == 0
    e_per_w = num_edges // _NW
    n_chunks = e_per_w // _C
    stride = n_pad // _NS
    assert n_pad % (_NS * _ZR) == 0

    mesh = plsc.VectorSubcoreMesh(core_axis_name="c", subcore_axis_name="s")

    @functools.partial(
        pl.kernel,
        mesh=mesh,
        out_type=jax.ShapeDtypeStruct((_NC, n_pad, _D), jnp.float32),
        scratch_types=[
            pltpu.VMEM((e_per_w,), jnp.int32),
            pltpu.VMEM((e_per_w,), jnp.int32),
            pltpu.VMEM((_C,), jnp.int32),
            pltpu.VMEM((_C,), jnp.int32),
            pltpu.VMEM((_C,), jnp.int32),
            pltpu.VMEM((_C,), jnp.int32),
            pltpu.VMEM((_C, _D), jnp.float32),
            pltpu.VMEM((_C, _D), jnp.float32),
            pltpu.VMEM((_ZR, _D), jnp.float32),
            pltpu.VMEM_SHARED((n_pad, _D), jnp.float32),
            pltpu.SemaphoreType.DMA,
            pltpu.SemaphoreType.DMA,
        ],
    )
    def seg(table, src, dst, out, src_all, dst_all, sc0, sc1, dc0, dc1,
            rows0, rows1, zbuf, acc, sem0, sem1):
        cid = lax.axis_index("c")
        sid = lax.axis_index("s")
        wid = cid * _NS + sid
        base = wid * e_per_w

        def zrow(r, _):
            for v in range(_D // 16):
                zbuf[r, pl.ds(v * 16, 16)] = jnp.zeros((16,), jnp.float32)
            return 0
        lax.fori_loop(0, _ZR, zrow, 0)

        rs = sid * stride

        def zfill(i, _):
            pltpu.sync_copy(zbuf, acc.at[pl.ds(rs + i * _ZR, _ZR)])
            return 0
        lax.fori_loop(0, stride // _ZR, zfill, 0)

        pltpu.sync_copy(src.at[pl.ds(base, e_per_w)], src_all)
        pltpu.sync_copy(dst.at[pl.ds(base, e_per_w)], dst_all)

        plsc.subcore_barrier()

        def stage_idx(chunk, sbuf, dbuf):
            off = chunk * _C
            for v in range(_C // 16):
                sbuf[pl.ds(v * 16, 16)] = src_all[pl.ds(off + v * 16, 16)]
                dbuf[pl.ds(v * 16, 16)] = dst_all[pl.ds(off + v * 16, 16)]

        stage_idx(0, sc0, dc0)
        pltpu.async_copy(table.at[sc0], rows0, sem0)

        n_pairs = (n_chunks - 1) // 2

        def pair(j, _):
            stage_idx(2 * j + 1, sc1, dc1)
            pltpu.async_copy(table.at[sc1], rows1, sem1)
            pltpu.make_async_copy(table.at[sc0], rows0, sem0).wait()
            pltpu.sync_copy(rows0, acc.at[dc0], add=True)
            stage_idx(2 * j + 2, sc0, dc0)
            pltpu.async_copy(table.at[sc0], rows0, sem0)
            pltpu.make_async_copy(table.at[sc1], rows1, sem1).wait()
            pltpu.sync_copy(rows1, acc.at[dc1], add=True)
            return 0

        lax.fori_loop(0, n_pairs, pair, 0)

        pltpu.make_async_copy(table.at[sc0], rows0, sem0).wait()
        pltpu.sync_copy(rows0, acc.at[dc0], add=True)
        if 2 * n_pairs + 1 < n_chunks:
            stage_idx(2 * n_pairs + 1, sc1, dc1)
            pltpu.async_copy(table.at[sc1], rows1, sem1).wait()
            pltpu.sync_copy(rows1, acc.at[dc1], add=True)

        plsc.subcore_barrier()

        pltpu.sync_copy(acc.at[pl.ds(rs, stride)],
                        out.at[cid, pl.ds(rs, stride)])

    return seg


@functools.cache
def _make_pool(n, k):
    def body(h_ref, s_ref, g_ref, hout_ref, alive_out_ref, ro_ref):
        h = h_ref[...]
        s = s_ref[...]
        g = g_ref[...]

        bits = lax.bitcast_convert_type(s, jnp.int32)
        mag = jnp.where(bits < 0, bits ^ jnp.int32(0x7FFFFFFF), bits)

        lo_t = jnp.int32(-2147483648)
        hi_t = jnp.int32(2147483647)
        for _ in range(32):
            x = lo_t ^ hi_t
            mid = (lo_t & hi_t) + (x >> 1) + (x & 1)
            cnt = jnp.sum((mag >= mid).astype(jnp.int32))
            ge = cnt >= k
            lo_t = jnp.where(ge, mid, lo_t)
            hi_t = jnp.where(ge, hi_t, mid - 1)
        T = lo_t

        r = k - jnp.sum((mag > T).astype(jnp.int32))
        tie = mag == T
        idx = lax.broadcasted_iota(jnp.int32, (n, 1), 0)
        lo = jnp.int32(0)
        hi = jnp.int32(n - 1)
        for _ in range(15):
            mid = (lo + hi) // 2
            c = jnp.sum((tie & (idx <= mid)).astype(jnp.int32))
            sel = c >= r
            hi = jnp.where(sel, mid, hi)
            lo = jnp.where(sel, lo, mid + 1)

        keep = (mag > T) | (tie & (idx <= hi))
        gfull = jnp.where(keep, g, 0.0)
        hout = h * gfull
        hout_ref[...] = hout
        alive_out_ref[...] = jnp.where(keep, 1.0, 0.0)
        gmax = jnp.max(jnp.where(keep, hout, -jnp.inf), axis=0)
        gmean = jnp.sum(hout, axis=0) * (1.0 / k)
        ro_ref[...] = jnp.concatenate([gmax, gmean])[None, :]

    return pl.pallas_call(
        body,
        out_shape=[
            jax.ShapeDtypeStruct((n, _D), jnp.float32),
            jax.ShapeDtypeStruct((n, 1), jnp.float32),
            jax.ShapeDtypeStruct((1, 2 * _D), jnp.float32),
        ],
    )


def _seg_sum(hW, src, dst):
    n = hW.shape[0]
    n_pad = _round_up(n, _NS * _ZR)
    table = jnp.pad(hW, ((0, n_pad - n), (0, 0)))
    out = _make_seg_sum(n_pad, src.shape[0])(table, src, dst)
    return out[0, :n] + out[1, :n]


def kernel(x, edge_index, batch, Wr1, Wl1, b1, p1, Wr2, Wl2, b2, p2,
           Wr3, Wl3, b3, p3, Wlin1, blin1, Wlin2, blin2, Wlin3, blin3):
    n = x.shape[0]
    src = edge_index[0]
    dst = edge_index[1]

    h = x
    alive = jnp.ones((n, 1), jnp.float32)
    k = n
    readouts = []
    for Wr, Wl, b, p in ((Wr1, Wl1, b1, p1), (Wr2, Wl2, b2, p2),
                         (Wr3, Wl3, b3, p3)):
        agg = _seg_sum(h @ Wl, src, dst)
        h = jax.nn.relu(h @ Wr + agg + b) * alive

        k = int(math.ceil(0.5 * k))
        score = (h @ p) / jnp.linalg.norm(p)
        score = jnp.where(alive[:, 0] > 0.5, score, -jnp.inf)
        h, alive, ro = _make_pool(n, k)(
            h, score[:, None], jnp.tanh(score)[:, None])
        readouts.append(ro)

    z = jnp.concatenate(readouts, axis=1)
    z = jax.nn.relu(z @ Wlin1 + blin1)
    z = jax.nn.relu(z @ Wlin2 + blin2)
    return jax.nn.log_softmax(z @ Wlin3 + blin3, axis=-1)


# speedup vs baseline: 1.1325x; 1.1325x over previous
import functools
import math

import jax
import jax.numpy as jnp
from jax import lax
from jax.experimental import pallas as pl
from jax.experimental.pallas import tpu as pltpu
from jax.experimental.pallas import tpu_sc as plsc

_D = 128
_NC = 2
_NS = 16
_NW = _NC * _NS
_C = 80
_ZR = 8


def _round_up(x, m):
    return (x + m - 1) // m * m


@functools.cache
def _make_seg_sum(n_pad, num_edges):
    assert num_edges % _NW == 0
    e_per_w = num_edges // _NW
    n_chunks = e_per_w // _C
    tail = e_per_w - n_chunks * _C
    assert tail % 16 == 0
    stride = n_pad // _NS
    assert n_pad % (_NS * _ZR) == 0

    mesh = plsc.VectorSubcoreMesh(core_axis_name="c", subcore_axis_name="s")

    @functools.partial(
        pl.kernel,
        mesh=mesh,
        out_type=jax.ShapeDtypeStruct((_NC, n_pad, _D), jnp.float32),
        scratch_types=[
            pltpu.VMEM((e_per_w,), jnp.int32),
            pltpu.VMEM((e_per_w,), jnp.int32),
            pltpu.VMEM((_C,), jnp.int32),
            pltpu.VMEM((_C,), jnp.int32),
            pltpu.VMEM((_C,), jnp.int32),
            pltpu.VMEM((_C,), jnp.int32),
            pltpu.VMEM((_C, _D), jnp.float32),
            pltpu.VMEM((_C, _D), jnp.float32),
            pltpu.VMEM((max(tail, 16),), jnp.int32),
            pltpu.VMEM((max(tail, 16),), jnp.int32),
            pltpu.VMEM((max(tail, 16), _D), jnp.float32),
            pltpu.VMEM((_ZR, _D), jnp.float32),
            pltpu.VMEM_SHARED((n_pad, _D), jnp.float32),
            pltpu.SemaphoreType.DMA,
            pltpu.SemaphoreType.DMA,
        ],
    )
    def seg(table, src, dst, out, src_all, dst_all, sc0, sc1, dc0, dc1,
            rows0, rows1, sct, dct, rowst, zbuf, acc, sem0, sem1):
        cid = lax.axis_index("c")
        sid = lax.axis_index("s")
        wid = cid * _NS + sid
        base = wid * e_per_w

        def zrow(r, _):
            for v in range(_D // 16):
                zbuf[r, pl.ds(v * 16, 16)] = jnp.zeros((16,), jnp.float32)
            return 0
        lax.fori_loop(0, _ZR, zrow, 0)

        rs = sid * stride

        def zfill(i, _):
            pltpu.sync_copy(zbuf, acc.at[pl.ds(rs + i * _ZR, _ZR)])
            return 0
        lax.fori_loop(0, stride // _ZR, zfill, 0)

        pltpu.sync_copy(src.at[pl.ds(base, e_per_w)], src_all)
        pltpu.sync_copy(dst.at[pl.ds(base, e_per_w)], dst_all)

        plsc.subcore_barrier()

        def stage_idx(chunk, sbuf, dbuf):
            off = chunk * _C
            for v in range(_C // 16):
                sbuf[pl.ds(v * 16, 16)] = src_all[pl.ds(off + v * 16, 16)]
                dbuf[pl.ds(v * 16, 16)] = dst_all[pl.ds(off + v * 16, 16)]

        stage_idx(0, sc0, dc0)
        pltpu.async_copy(table.at[sc0], rows0, sem0)

        n_pairs = (n_chunks - 1) // 2

        def pair(j, _):
            stage_idx(2 * j + 1, sc1, dc1)
            pltpu.async_copy(table.at[sc1], rows1, sem1)
            pltpu.make_async_copy(table.at[sc0], rows0, sem0).wait()
            pltpu.sync_copy(rows0, acc.at[dc0], add=True)
            stage_idx(2 * j + 2, sc0, dc0)
            pltpu.async_copy(table.at[sc0], rows0, sem0)
            pltpu.make_async_copy(table.at[sc1], rows1, sem1).wait()
            pltpu.sync_copy(rows1, acc.at[dc1], add=True)
            return 0

        lax.fori_loop(0, n_pairs, pair, 0)

        pltpu.make_async_copy(table.at[sc0], rows0, sem0).wait()
        pltpu.sync_copy(rows0, acc.at[dc0], add=True)
        if 2 * n_pairs + 1 < n_chunks:
            stage_idx(2 * n_pairs + 1, sc1, dc1)
            pltpu.async_copy(table.at[sc1], rows1, sem1).wait()
            pltpu.sync_copy(rows1, acc.at[dc1], add=True)

        if tail:
            off = n_chunks * _C
            for v in range(tail // 16):
                sct[pl.ds(v * 16, 16)] = src_all[pl.ds(off + v * 16, 16)]
                dct[pl.ds(v * 16, 16)] = dst_all[pl.ds(off + v * 16, 16)]
            pltpu.async_copy(table.at[sct], rowst, sem0).wait()
            pltpu.sync_copy(rowst, acc.at[dct], add=True)

        plsc.subcore_barrier()

        pltpu.sync_copy(acc.at[pl.ds(rs, stride)],
                        out.at[cid, pl.ds(rs, stride)])

    return seg


def _seg_sum(hW, src, dst):
    n = hW.shape[0]
    n_pad = _round_up(n, _NS * _ZR)
    table = jnp.pad(hW, ((0, n_pad - n), (0, 0)))
    out = _make_seg_sum(n_pad, src.shape[0])(table, src, dst)
    return out[0, :n] + out[1, :n]


def kernel(x, edge_index, batch, Wr1, Wl1, b1, p1, Wr2, Wl2, b2, p2,
           Wr3, Wl3, b3, p3, Wlin1, blin1, Wlin2, blin2, Wlin3, blin3):
    n = x.shape[0]
    src = edge_index[0]
    dst = edge_index[1]

    h = x
    alive = jnp.ones((n,), jnp.bool_)
    k = n
    readouts = []
    for Wr, Wl, b, p in ((Wr1, Wl1, b1, p1), (Wr2, Wl2, b2, p2),
                         (Wr3, Wl3, b3, p3)):
        agg = _seg_sum(h @ Wl, src, dst)
        h = jnp.where(alive[:, None], jax.nn.relu(h @ Wr + agg + b), 0.0)

        k = int(math.ceil(0.5 * k))
        score = (h @ p) / jnp.linalg.norm(p)
        score = jnp.where(alive, score, -jnp.inf)
        _, perm = lax.top_k(score, k)
        alive = jnp.zeros((n,), jnp.bool_).at[perm].set(True)
        h = jnp.where(alive[:, None], h * jnp.tanh(score)[:, None], 0.0)

        gmax = jnp.max(jnp.where(alive[:, None], h, -jnp.inf), axis=0)
        gmean = jnp.sum(h, axis=0) / k
        readouts.append(jnp.concatenate([gmax, gmean])[None, :])

    z = jnp.concatenate(readouts, axis=1)
    z = jax.nn.relu(z @ Wlin1 + blin1)
    z = jax.nn.relu(z @ Wlin2 + blin2)
    return jax.nn.log_softmax(z @ Wlin3 + blin3, axis=-1)


# speedup vs baseline: 1.1670x; 1.0305x over previous
import functools
import math

import jax
import jax.numpy as jnp
from jax import lax
from jax.experimental import pallas as pl
from jax.experimental.pallas import tpu as pltpu
from jax.experimental.pallas import tpu_sc as plsc

_D = 128
_NC = 2
_NS = 16
_NW = _NC * _NS
_C = 80
_ZR = 32


def _round_up(x, m):
    return (x + m - 1) // m * m


@functools.cache
def _make_seg_sum(n_pad, num_edges):
    assert num_edges % _NW == 0
    e_per_w = num_edges // _NW
    n_chunks = e_per_w // _C
    tail = e_per_w - n_chunks * _C
    assert tail % 16 == 0
    stride = n_pad // _NS
    assert n_pad % _NS == 0 and stride % 8 == 0
    zfull = stride // _ZR
    ztail = stride - zfull * _ZR

    mesh = plsc.VectorSubcoreMesh(core_axis_name="c", subcore_axis_name="s")

    @functools.partial(
        pl.kernel,
        mesh=mesh,
        out_type=jax.ShapeDtypeStruct((_NC, n_pad, _D), jnp.float32),
        scratch_types=[
            pltpu.VMEM((e_per_w,), jnp.int32),
            pltpu.VMEM((e_per_w,), jnp.int32),
            pltpu.VMEM((_C,), jnp.int32),
            pltpu.VMEM((_C,), jnp.int32),
            pltpu.VMEM((_C,), jnp.int32),
            pltpu.VMEM((_C,), jnp.int32),
            pltpu.VMEM((_C, _D), jnp.float32),
            pltpu.VMEM((_C, _D), jnp.float32),
            pltpu.VMEM((max(tail, 16),), jnp.int32),
            pltpu.VMEM((max(tail, 16),), jnp.int32),
            pltpu.VMEM((max(tail, 16), _D), jnp.float32),
            pltpu.VMEM((_ZR, _D), jnp.float32),
            pltpu.VMEM_SHARED((n_pad, _D), jnp.float32),
            pltpu.SemaphoreType.DMA,
            pltpu.SemaphoreType.DMA,
        ],
    )
    def seg(table, src, dst, out, src_all, dst_all, sc0, sc1, dc0, dc1,
            rows0, rows1, sct, dct, rowst, zbuf, acc, sem0, sem1):
        cid = lax.axis_index("c")
        sid = lax.axis_index("s")
        wid = cid * _NS + sid
        base = wid * e_per_w

        def zrow(r, _):
            for v in range(_D // 16):
                zbuf[r, pl.ds(v * 16, 16)] = jnp.zeros((16,), jnp.float32)
            return 0
        lax.fori_loop(0, _ZR, zrow, 0)

        rs = sid * stride

        def zfire(i, _):
            pltpu.async_copy(zbuf, acc.at[pl.ds(rs + i * _ZR, _ZR)], sem0)
            return 0
        lax.fori_loop(0, zfull, zfire, 0)
        if ztail:
            pltpu.async_copy(zbuf.at[pl.ds(0, ztail)],
                             acc.at[pl.ds(rs + zfull * _ZR, ztail)], sem0)

        pltpu.sync_copy(src.at[pl.ds(base, e_per_w)], src_all)
        pltpu.sync_copy(dst.at[pl.ds(base, e_per_w)], dst_all)

        def zdrain(i, _):
            pltpu.make_async_copy(zbuf, acc.at[pl.ds(rs, _ZR)], sem0).wait()
            return 0
        lax.fori_loop(0, zfull, zdrain, 0)
        if ztail:
            pltpu.make_async_copy(zbuf.at[pl.ds(0, ztail)],
                                  acc.at[pl.ds(rs, ztail)], sem0).wait()

        plsc.subcore_barrier()

        def stage_idx(chunk, sbuf, dbuf):
            off = chunk * _C
            for v in range(_C // 16):
                sbuf[pl.ds(v * 16, 16)] = src_all[pl.ds(off + v * 16, 16)]
                dbuf[pl.ds(v * 16, 16)] = dst_all[pl.ds(off + v * 16, 16)]

        stage_idx(0, sc0, dc0)
        pltpu.async_copy(table.at[sc0], rows0, sem0)

        n_pairs = (n_chunks - 1) // 2

        def pair(j, _):
            stage_idx(2 * j + 1, sc1, dc1)
            pltpu.async_copy(table.at[sc1], rows1, sem1)
            pltpu.make_async_copy(table.at[sc0], rows0, sem0).wait()
            pltpu.sync_copy(rows0, acc.at[dc0], add=True)
            stage_idx(2 * j + 2, sc0, dc0)
            pltpu.async_copy(table.at[sc0], rows0, sem0)
            pltpu.make_async_copy(table.at[sc1], rows1, sem1).wait()
            pltpu.sync_copy(rows1, acc.at[dc1], add=True)
            return 0

        lax.fori_loop(0, n_pairs, pair, 0)

        pltpu.make_async_copy(table.at[sc0], rows0, sem0).wait()
        pltpu.sync_copy(rows0, acc.at[dc0], add=True)
        if 2 * n_pairs + 1 < n_chunks:
            stage_idx(2 * n_pairs + 1, sc1, dc1)
            pltpu.async_copy(table.at[sc1], rows1, sem1).wait()
            pltpu.sync_copy(rows1, acc.at[dc1], add=True)

        if tail:
            off = n_chunks * _C
            for v in range(tail // 16):
                sct[pl.ds(v * 16, 16)] = src_all[pl.ds(off + v * 16, 16)]
                dct[pl.ds(v * 16, 16)] = dst_all[pl.ds(off + v * 16, 16)]
            pltpu.async_copy(table.at[sct], rowst, sem0).wait()
            pltpu.sync_copy(rowst, acc.at[dct], add=True)

        plsc.subcore_barrier()

        pltpu.sync_copy(acc.at[pl.ds(rs, stride)],
                        out.at[cid, pl.ds(rs, stride)])

    return seg


def _seg_sum(hW, src, dst):
    n = hW.shape[0]
    n_pad = _round_up(n, _NS * _ZR)
    table = jnp.pad(hW, ((0, n_pad - n), (0, 0)))
    out = _make_seg_sum(n_pad, src.shape[0])(table, src, dst)
    return out[0, :n] + out[1, :n]


def kernel(x, edge_index, batch, Wr1, Wl1, b1, p1, Wr2, Wl2, b2, p2,
           Wr3, Wl3, b3, p3, Wlin1, blin1, Wlin2, blin2, Wlin3, blin3):
    n = x.shape[0]
    src = edge_index[0]
    dst = edge_index[1]

    h = x
    alive = jnp.ones((n,), jnp.bool_)
    k = n
    readouts = []
    for Wr, Wl, b, p in ((Wr1, Wl1, b1, p1), (Wr2, Wl2, b2, p2),
                         (Wr3, Wl3, b3, p3)):
        agg = _seg_sum(h @ Wl, src, dst)
        h = jnp.where(alive[:, None], jax.nn.relu(h @ Wr + agg + b), 0.0)

        k = int(math.ceil(0.5 * k))
        score = (h @ p) / jnp.linalg.norm(p)
        score = jnp.where(alive, score, -jnp.inf)
        _, perm = lax.top_k(score, k)
        alive = jnp.zeros((n,), jnp.bool_).at[perm].set(True)
        h = jnp.where(alive[:, None], h * jnp.tanh(score)[:, None], 0.0)

        gmax = jnp.max(jnp.where(alive[:, None], h, -jnp.inf), axis=0)
        gmean = jnp.sum(h, axis=0) / k
        readouts.append(jnp.concatenate([gmax, gmean])[None, :])

    z = jnp.concatenate(readouts, axis=1)
    z = jax.nn.relu(z @ Wlin1 + blin1)
    z = jax.nn.relu(z @ Wlin2 + blin2)
    return jax.nn.log_softmax(z @ Wlin3 + blin3, axis=-1)
